# Initial kernel scaffold; baseline (speedup 1.0000x reference)
import jax
import jax.numpy as jnp
from jax import lax
from jax.experimental import pallas as pl
from jax.experimental.pallas import tpu as pltpu
from jax.experimental.pallas import tpu_sc as plsc

N = 10000
D = 128
NC = 2
NS = 16
NW = NC * NS

CHUNK_ROWS = 4
CHUNK_E = CHUNK_ROWS * 128
N_CHUNKS = 20
EW_ROWS = CHUNK_ROWS * N_CHUNKS
E_PAD = NW * EW_ROWS * 128
AGG_ROWS = N + 16
TPW = N // NS
ZPT = AGG_ROWS // NS

AW_ROWS = 3
AW_NODES = AW_ROWS * 128
NA_PAD = NW * AW_NODES


def _atom_body(xT_hbm, emb_hbm, out_hbm, idx_v, h0_v, sem):
    core = lax.axis_index("c")
    sub = lax.axis_index("s")
    w = sub * NC + core
    rb = w * AW_ROWS
    for f in range(9):
        pltpu.sync_copy(xT_hbm.at[f, pl.ds(rb, AW_ROWS)], idx_v)
        if f:
            for j in range(AW_ROWS):
                for i in range(8):
                    s = pl.ds(16 * i, 16)
                    idx_v[j, s] = idx_v[j, s] + (64 * f)
        cps = [
            pltpu.async_copy(emb_hbm.at[idx_v.at[j]],
                             h0_v.at[pl.ds(128 * j, 128)], sem, add=(f > 0))
            for j in range(AW_ROWS)
        ]
        for c in cps:
            c.wait()
    pltpu.sync_copy(h0_v, out_hbm.at[pl.ds(w * AW_NODES, AW_NODES)])


def _mp_body(h_hbm, src_hbm, dst_hbm, ea0_hbm, ea1_hbm, ea2_hbm, ee_hbm,
             z_hbm, out_hbm, src_v, dst_v, ea_v, combo_v, msg_v, agg_sp,
             sem, sem2):
    core = lax.axis_index("c")
    sub = lax.axis_index("s")
    w = sub * NC + core
    pltpu.sync_copy(z_hbm, agg_sp.at[pl.ds(sub * ZPT, ZPT)])
    plsc.subcore_barrier()

    base_rows = w * EW_ROWS

    def chunk(k, carry):
        rb = base_rows + k * CHUNK_ROWS
        pltpu.sync_copy(src_hbm.at[pl.ds(rb, CHUNK_ROWS)], src_v)
        pltpu.sync_copy(dst_hbm.at[pl.ds(rb, CHUNK_ROWS)], dst_v)
        pltpu.sync_copy(ea0_hbm.at[pl.ds(rb, CHUNK_ROWS)], ea_v.at[0])
        pltpu.sync_copy(ea1_hbm.at[pl.ds(rb, CHUNK_ROWS)], ea_v.at[1])
        pltpu.sync_copy(ea2_hbm.at[pl.ds(rb, CHUNK_ROWS)], ea_v.at[2])
        for j in range(CHUNK_ROWS):
            for i in range(8):
                s = pl.ds(16 * i, 16)
                combo_v[j, s] = (ea_v[0, j, s] + ea_v[1, j, s] * 4
                                 + ea_v[2, j, s] * 16)
        cps = [
            pltpu.async_copy(ee_hbm.at[combo_v.at[j]],
                             msg_v.at[pl.ds(128 * j, 128)], sem)
            for j in range(CHUNK_ROWS)
        ]
        for c in cps:
            c.wait()
        cps = [
            pltpu.async_copy(h_hbm.at[src_v.at[j]],
                             msg_v.at[pl.ds(128 * j, 128)], sem, add=True)
            for j in range(CHUNK_ROWS)
        ]
        for c in cps:
            c.wait()

        def relu_row(e, c2):
            for i in range(8):
                s = pl.ds(16 * i, 16)
                msg_v[e, s] = jnp.maximum(msg_v[e, s], 0.0)
            return c2

        lax.fori_loop(0, CHUNK_E, relu_row, 0)
        cps = [
            pltpu.async_copy(msg_v.at[pl.ds(128 * j, 128)],
                             agg_sp.at[dst_v.at[j]], sem2, add=True)
            for j in range(CHUNK_ROWS)
        ]
        for c in cps:
            c.wait()
        return carry

    lax.fori_loop(0, N_CHUNKS, chunk, 0)
    plsc.subcore_barrier()
    pltpu.sync_copy(agg_sp.at[pl.ds(sub * TPW, TPW)],
                    out_hbm.at[core, pl.ds(sub * TPW, TPW)])


def _mlp(h, agg, W1l, b1l, g1l, bt1l, W2l, b2l, g2l, bt2l, scale, final):
    def body(h_ref, a_ref, W1_ref, b1_ref, g1_ref, bt1_ref, W2_ref, b2_ref,
             g2_ref, bt2_ref, s_ref, o_ref):
        x = h_ref[...] * s_ref[0, 0] + a_ref[0] + a_ref[1]
        z = jnp.dot(x, W1_ref[...], preferred_element_type=jnp.float32)
        z = z + b1_ref[...]
        m = jnp.mean(z, axis=0, keepdims=True)
        zc = z - m
        v = jnp.mean(zc * zc, axis=0, keepdims=True)
        z = zc * jax.lax.rsqrt(v + 1e-5) * g1_ref[...] + bt1_ref[...]
        z = jnp.maximum(z, 0.0)
        z2 = jnp.dot(z, W2_ref[...], preferred_element_type=jnp.float32)
        z2 = z2 + b2_ref[...]
        m2 = jnp.mean(z2, axis=0, keepdims=True)
        zc2 = z2 - m2
        v2 = jnp.mean(zc2 * zc2, axis=0, keepdims=True)
        z2 = zc2 * jax.lax.rsqrt(v2 + 1e-5) * g2_ref[...] + bt2_ref[...]
        o_ref[...] = z2 if final else jnp.maximum(z2, 0.0)

    return pl.pallas_call(
        body, out_shape=jax.ShapeDtypeStruct((N, D), jnp.float32))(
            h, agg, W1l, b1l.reshape(1, -1), g1l.reshape(1, -1),
            bt1l.reshape(1, -1), W2l, b2l.reshape(1, -1), g2l.reshape(1, -1),
            bt2l.reshape(1, -1), scale)


def kernel(x, edge_index, edge_attr, batch, atom_emb, bond_emb, W1, b1, g1,
           bt1, W2, b2, g2, bt2, eps):
    xT = jnp.transpose(x).astype(jnp.int32)
    xT = jnp.pad(xT, ((0, 0), (0, NA_PAD - N))).reshape(9, NA_PAD // 128, 128)
    emb_flat = atom_emb.reshape(9 * 64, D).astype(jnp.float32)

    E = edge_index.shape[1]
    pad = E_PAD - E
    src2d = jnp.pad(edge_index[0].astype(jnp.int32),
                    (0, pad)).reshape(E_PAD // 128, 128)
    dst2d = jnp.pad(edge_index[1].astype(jnp.int32), (0, pad),
                    constant_values=N).reshape(E_PAD // 128, 128)
    eaT = edge_attr.astype(jnp.int32).T
    ea2d = [jnp.pad(eaT[f], (0, pad)).reshape(E_PAD // 128, 128)
            for f in range(3)]

    cc = jnp.arange(64)
    i0, i1, i2 = cc % 4, (cc // 4) % 4, (cc // 16) % 4
    zeros_z = jnp.zeros((ZPT, D), jnp.float32)

    mesh = plsc.VectorSubcoreMesh(core_axis_name="c", subcore_axis_name="s",
                                  num_cores=NC, num_subcores=NS)

    atom_call = pl.kernel(
        _atom_body,
        out_type=jax.ShapeDtypeStruct((NA_PAD, D), jnp.float32),
        mesh=mesh,
        scratch_types=[
            pltpu.VMEM((AW_ROWS, 128), jnp.int32),
            pltpu.VMEM((AW_NODES, D), jnp.float32),
            pltpu.SemaphoreType.DMA,
        ])
    h = atom_call(xT, emb_flat)[:N]

    mp_call = pl.kernel(
        _mp_body,
        out_type=jax.ShapeDtypeStruct((NC, N, D), jnp.float32),
        mesh=mesh,
        scratch_types=[
            pltpu.VMEM((CHUNK_ROWS, 128), jnp.int32),
            pltpu.VMEM((CHUNK_ROWS, 128), jnp.int32),
            pltpu.VMEM((3, CHUNK_ROWS, 128), jnp.int32),
            pltpu.VMEM((CHUNK_ROWS, 128), jnp.int32),
            pltpu.VMEM((CHUNK_E, D), jnp.float32),
            pltpu.VMEM_SHARED((AGG_ROWS, D), jnp.float32),
            pltpu.SemaphoreType.DMA,
            pltpu.SemaphoreType.DMA,
        ])

    L = W1.shape[0]
    for l in range(L):
        ee = (bond_emb[l, 0, :4][i0] + bond_emb[l, 1, :4][i1]
              + bond_emb[l, 2, :4][i2]).astype(jnp.float32)
        agg = mp_call(h, src2d, dst2d, ea2d[0], ea2d[1], ea2d[2], ee, zeros_z)
        scale = (1.0 + eps[l]).reshape(1, 1).astype(jnp.float32)
        h = _mlp(h, agg, W1[l], b1[l], g1[l], bt1[l], W2[l], b2[l], g2[l],
                 bt2[l], scale, final=(l == L - 1))
    return h


# speedup vs baseline: 2.5795x; 2.5795x over previous
import jax
import jax.numpy as jnp
from jax import lax
from jax.experimental import pallas as pl
from jax.experimental.pallas import tpu as pltpu
from jax.experimental.pallas import tpu_sc as plsc

N = 10000
D = 128
NC = 2
NS = 16
NW = NC * NS

CHUNK_ROWS = 8
MSG_ROWS = 256
HALF_E = 512
N_CHUNKS = 10
EW_ROWS = CHUNK_ROWS * N_CHUNKS
E_PAD = NW * EW_ROWS * 128
AGG_ROWS = 10240
TPW = AGG_ROWS // NS

NA_PAD = 10240
AB_ROWS = 8
NA_W = NA_PAD // (AB_ROWS * 128)


def _atom_body(xT_hbm, emb_hbm, out_hbm, idx_v, h0_v, sem):
    core = lax.axis_index("c")
    sub = lax.axis_index("s")
    w = sub * NC + core

    @pl.when(w < NA_W)
    def _():
        rb = w * AB_ROWS
        for t in range(2):
            for f in range(9):
                pltpu.sync_copy(xT_hbm.at[f, pl.ds(rb, AB_ROWS)], idx_v)
                if f:
                    for j in range(4 * t, 4 * t + 4):
                        for i in range(8):
                            s = pl.ds(16 * i, 16)
                            idx_v[j, s] = idx_v[j, s] + (64 * f)
                cps = [
                    pltpu.async_copy(emb_hbm.at[idx_v.at[4 * t + j]],
                                     h0_v.at[pl.ds(128 * j, 128)], sem,
                                     add=(f > 0))
                    for j in range(4)
                ]
                for c in cps:
                    c.wait()
            pltpu.sync_copy(
                h0_v, out_hbm.at[pl.ds(w * 1024 + 512 * t, HALF_E)])


def _mp_body(h_hbm, src_hbm, dst_hbm, ea0_hbm, ea1_hbm, ea2_hbm, ee_hbm,
             z_hbm, out_hbm, src_v, dst_v, ea_v, combo_v, msg_v, agg_sp,
             sem, sem2):
    core = lax.axis_index("c")
    sub = lax.axis_index("s")
    w = sub * NC + core
    pltpu.sync_copy(z_hbm, agg_sp.at[pl.ds(sub * TPW, TPW)])
    plsc.subcore_barrier()

    base_rows = w * EW_ROWS

    def chunk(k, carry):
        rb = base_rows + k * CHUNK_ROWS
        pltpu.sync_copy(src_hbm.at[pl.ds(rb, CHUNK_ROWS)], src_v)
        pltpu.sync_copy(dst_hbm.at[pl.ds(rb, CHUNK_ROWS)], dst_v)
        pltpu.sync_copy(ea0_hbm.at[pl.ds(rb, CHUNK_ROWS)], ea_v.at[0])
        pltpu.sync_copy(ea1_hbm.at[pl.ds(rb, CHUNK_ROWS)], ea_v.at[1])
        pltpu.sync_copy(ea2_hbm.at[pl.ds(rb, CHUNK_ROWS)], ea_v.at[2])
        for j in range(CHUNK_ROWS):
            for i in range(8):
                s = pl.ds(16 * i, 16)
                combo_v[j, s] = (ea_v[0, j, s] + ea_v[1, j, s] * 4
                                 + ea_v[2, j, s] * 16)
        for t in range(4):
            cps = [
                pltpu.async_copy(ee_hbm.at[combo_v.at[2 * t + j]],
                                 msg_v.at[pl.ds(128 * j, 128)], sem)
                for j in range(2)
            ]
            for c in cps:
                c.wait()
            cps = [
                pltpu.async_copy(h_hbm.at[src_v.at[2 * t + j]],
                                 msg_v.at[pl.ds(128 * j, 128)], sem,
                                 add=True)
                for j in range(2)
            ]
            for c in cps:
                c.wait()

            def relu_row(e, c2):
                for i in range(8):
                    s = pl.ds(16 * i, 16)
                    msg_v[e, s] = jnp.maximum(msg_v[e, s], 0.0)
                return c2

            lax.fori_loop(0, MSG_ROWS, relu_row, 0)
            cps = [
                pltpu.async_copy(msg_v.at[pl.ds(128 * j, 128)],
                                 agg_sp.at[dst_v.at[2 * t + j]], sem2,
                                 add=True)
                for j in range(2)
            ]
            for c in cps:
                c.wait()
        return carry

    lax.fori_loop(0, N_CHUNKS, chunk, 0)
    plsc.subcore_barrier()
    pltpu.sync_copy(agg_sp.at[pl.ds(sub * TPW, TPW)],
                    out_hbm.at[core, pl.ds(sub * TPW, TPW)])


def _mlp(h, agg, W1l, b1l, g1l, bt1l, W2l, b2l, g2l, bt2l, scale, final):
    def body(h_ref, a_ref, W1_ref, b1_ref, g1_ref, bt1_ref, W2_ref, b2_ref,
             g2_ref, bt2_ref, s_ref, o_ref):
        x = h_ref[...] * s_ref[0, 0] + a_ref[0, :N, :] + a_ref[1, :N, :]
        z = jnp.dot(x, W1_ref[...], preferred_element_type=jnp.float32)
        z = z + b1_ref[...]
        m = jnp.mean(z, axis=0, keepdims=True)
        zc = z - m
        v = jnp.mean(zc * zc, axis=0, keepdims=True)
        z = zc * jax.lax.rsqrt(v + 1e-5) * g1_ref[...] + bt1_ref[...]
        z = jnp.maximum(z, 0.0)
        z2 = jnp.dot(z, W2_ref[...], preferred_element_type=jnp.float32)
        z2 = z2 + b2_ref[...]
        m2 = jnp.mean(z2, axis=0, keepdims=True)
        zc2 = z2 - m2
        v2 = jnp.mean(zc2 * zc2, axis=0, keepdims=True)
        z2 = zc2 * jax.lax.rsqrt(v2 + 1e-5) * g2_ref[...] + bt2_ref[...]
        o_ref[...] = z2 if final else jnp.maximum(z2, 0.0)

    return pl.pallas_call(
        body, out_shape=jax.ShapeDtypeStruct((N, D), jnp.float32))(
            h, agg, W1l, b1l.reshape(1, -1), g1l.reshape(1, -1),
            bt1l.reshape(1, -1), W2l, b2l.reshape(1, -1), g2l.reshape(1, -1),
            bt2l.reshape(1, -1), scale)


def kernel(x, edge_index, edge_attr, batch, atom_emb, bond_emb, W1, b1, g1,
           bt1, W2, b2, g2, bt2, eps):
    xT = jnp.transpose(x).astype(jnp.int32)
    xT = jnp.pad(xT, ((0, 0), (0, NA_PAD - N))).reshape(9, NA_PAD // 128, 128)
    emb_flat = atom_emb.reshape(9 * 64, D).astype(jnp.float32)

    E = edge_index.shape[1]
    pad = E_PAD - E
    src2d = jnp.pad(edge_index[0].astype(jnp.int32),
                    (0, pad)).reshape(E_PAD // 128, 128)
    dst2d = jnp.pad(edge_index[1].astype(jnp.int32), (0, pad),
                    constant_values=N).reshape(E_PAD // 128, 128)
    eaT = edge_attr.astype(jnp.int32).T
    ea2d = [jnp.pad(eaT[f], (0, pad)).reshape(E_PAD // 128, 128)
            for f in range(3)]

    cc = jnp.arange(64)
    i0, i1, i2 = cc % 4, (cc // 4) % 4, (cc // 16) % 4
    zeros_z = jnp.zeros((TPW, D), jnp.float32)

    mesh = plsc.VectorSubcoreMesh(core_axis_name="c", subcore_axis_name="s",
                                  num_cores=NC, num_subcores=NS)

    atom_call = pl.kernel(
        _atom_body,
        out_type=jax.ShapeDtypeStruct((NA_PAD, D), jnp.float32),
        mesh=mesh,
        scratch_types=[
            pltpu.VMEM((AB_ROWS, 128), jnp.int32),
            pltpu.VMEM((HALF_E, D), jnp.float32),
            pltpu.SemaphoreType.DMA,
        ])
    h = atom_call(xT, emb_flat)[:N]

    mp_call = pl.kernel(
        _mp_body,
        out_type=jax.ShapeDtypeStruct((NC, AGG_ROWS, D), jnp.float32),
        mesh=mesh,
        scratch_types=[
            pltpu.VMEM((CHUNK_ROWS, 128), jnp.int32),
            pltpu.VMEM((CHUNK_ROWS, 128), jnp.int32),
            pltpu.VMEM((3, CHUNK_ROWS, 128), jnp.int32),
            pltpu.VMEM((CHUNK_ROWS, 128), jnp.int32),
            pltpu.VMEM((MSG_ROWS, D), jnp.float32),
            pltpu.VMEM_SHARED((AGG_ROWS, D), jnp.float32),
            pltpu.SemaphoreType.DMA,
            pltpu.SemaphoreType.DMA,
        ])

    L = W1.shape[0]
    for l in range(L):
        ee = (bond_emb[l, 0, :4][i0] + bond_emb[l, 1, :4][i1]
              + bond_emb[l, 2, :4][i2]).astype(jnp.float32)
        agg = mp_call(h, src2d, dst2d, ea2d[0], ea2d[1], ea2d[2], ee, zeros_z)
        scale = (1.0 + eps[l]).reshape(1, 1).astype(jnp.float32)
        h = _mlp(h, agg, W1[l], b1[l], g1[l], bt1[l], W2[l], b2[l], g2[l],
                 bt2[l], scale, final=(l == L - 1))
    return h
